# baseline (speedup 1.0000x reference)
import functools
import math

import jax
import jax.numpy as jnp
from jax import lax
from jax.experimental import pallas as pl
from jax.experimental.pallas import tpu as pltpu
from jax.experimental.pallas import tpu_sc as plsc

_SIGMA_C = 1.0
_RHO = 0.99
_SIGMA_S = _SIGMA_C * 1.6
_MAX_T = math.ceil(_SIGMA_S * 3)


def _gauss(x, sigma):
    return math.exp(-x * x / (2.0 * sigma * sigma)) / (math.sqrt(2.0 * math.pi) * sigma)


_W = {t: _gauss(t, _SIGMA_C) - _RHO * _gauss(t, _SIGMA_S)
      for t in range(-_MAX_T, _MAX_T + 1)}
_TOTAL_W = sum(_W.values())
_TAPS = [t for t in range(-_MAX_T, _MAX_T + 1) if t != 0]
_NT = len(_TAPS)

_B, _Y, _X = 16, 512, 512
_N = _Y * _X
_NC, _NS, _L = 2, 16, 16
_PX = _N // _NS
_CH = 2048
_NV = _CH // _L
_GS = 128
_NG = _CH // _GS
_MAGIC = 2.0 ** 23
_UNROLL = 8


def _dog_body(img_hbm, etf_hbm, out_hbm,
              plane_sh, ety_v, etx_v, iyf_v, ixf_v,
              idx_a, idx_b, val_a, val_b, val_c, val_0, acc_v,
              sem_a, sem_b, sem_0):
    c = lax.axis_index("c")
    s = lax.axis_index("s")
    nb = _B // _NC
    idx_bufs, val_bufs, sems = [idx_a, idx_b], [val_a, val_b, val_c], [sem_a, sem_b]

    def batch_body(k, carry):
        b = c * nb + k
        pltpu.sync_copy(img_hbm.at[b, pl.ds(s * _PX, _PX)],
                        plane_sh.at[pl.ds(s * _PX, _PX)])
        plsc.subcore_barrier()

        def chunk_body(ch, carry2):
            base = s * _PX + ch * _CH
            pltpu.sync_copy(etf_hbm.at[b, 1, pl.ds(base, _CH)], ety_v)
            pltpu.sync_copy(etf_hbm.at[b, 0, pl.ds(base, _CH)], etx_v)

            @plsc.parallel_loop(0, _NV, 1, unroll=_UNROLL)
            def coord_body(v):
                sl = pl.ds(v * _L, _L)
                p = base + v * _L + lax.iota(jnp.int32, _L)
                iyf_v[sl] = jnp.right_shift(p, 9).astype(jnp.float32)
                ixf_v[sl] = jnp.bitwise_and(p, _X - 1).astype(jnp.float32)

            def tap_idx(v, t, idx_ref):
                sl = pl.ds(v * _L, _L)
                tf = jnp.float32(t)
                mg = jnp.float32(_MAGIC)
                py = iyf_v[sl] - ety_v[sl] * tf
                px = ixf_v[sl] + etx_v[sl] * tf
                py = jnp.minimum(jnp.maximum(py, jnp.float32(0.0)),
                                 jnp.float32(_Y - 1))
                px = jnp.minimum(jnp.maximum(px, jnp.float32(0.0)),
                                 jnp.float32(_X - 1))
                pyr = (py + mg) - mg
                pxr = (px + mg) - mg
                flat = pyr * jnp.float32(_X) + pxr
                idx_ref[sl] = flat.astype(jnp.int32)

            def compute_idx(t, idx_ref):
                @plsc.parallel_loop(0, _NV, 1, unroll=_UNROLL)
                def _(v):
                    tap_idx(v, t, idx_ref)

            def fire(idx_ref, val_ref, sem):
                return [pltpu.async_copy(
                    plane_sh.at[idx_ref.at[pl.ds(j * _GS, _GS)]],
                    val_ref.at[pl.ds(j * _GS, _GS)], sem)
                    for j in range(_NG)]

            cp0 = pltpu.async_copy(plane_sh.at[pl.ds(base, _CH)], val_0, sem_0)

            compute_idx(_TAPS[0], idx_bufs[0])
            inflight = {0: fire(idx_bufs[0], val_bufs[0], sems[0])}
            compute_idx(_TAPS[1], idx_bufs[1])
            inflight[1] = fire(idx_bufs[1], val_bufs[1], sems[1])

            for kk in range(_NT):
                for cp in inflight.pop(kk % 2):
                    cp.wait()
                w = jnp.float32(_W[_TAPS[kk]] / _TOTAL_W)
                val_ref = val_bufs[kk % 3]
                has_next = kk + 2 < _NT
                last = kk == _NT - 1
                if last:
                    cp0.wait()
                w0 = jnp.float32(_W[0] / _TOTAL_W)

                @plsc.parallel_loop(0, _NV, 1, unroll=_UNROLL)
                def merged(v):
                    sl = pl.ds(v * _L, _L)
                    if has_next:
                        tap_idx(v, _TAPS[kk + 2], idx_bufs[kk % 2])
                    upd = val_ref[sl] * w
                    if kk == 0:
                        acc_v[sl] = upd
                    elif last:
                        acc_v[sl] = acc_v[sl] + upd + val_0[sl] * w0
                    else:
                        acc_v[sl] = acc_v[sl] + upd

                if has_next:
                    inflight[kk % 2] = fire(idx_bufs[kk % 2],
                                            val_bufs[(kk + 2) % 3],
                                            sems[kk % 2])

            pltpu.sync_copy(acc_v, out_hbm.at[b, pl.ds(base, _CH)])
            return 0

        lax.fori_loop(0, _PX // _CH, chunk_body, 0)
        plsc.subcore_barrier()
        return 0

    lax.fori_loop(0, nb, batch_body, 0)


_dog_call = functools.partial(
    pl.kernel,
    out_type=jax.ShapeDtypeStruct((_B, _N), jnp.float32),
    mesh=plsc.VectorSubcoreMesh(core_axis_name="c", subcore_axis_name="s"),
    scratch_types=[
        pltpu.VMEM_SHARED((_N,), jnp.float32),
        pltpu.VMEM((_CH,), jnp.float32),
        pltpu.VMEM((_CH,), jnp.float32),
        pltpu.VMEM((_CH,), jnp.float32),
        pltpu.VMEM((_CH,), jnp.float32),
        pltpu.VMEM((_CH,), jnp.int32),
        pltpu.VMEM((_CH,), jnp.int32),
        pltpu.VMEM((_CH,), jnp.float32),
        pltpu.VMEM((_CH,), jnp.float32),
        pltpu.VMEM((_CH,), jnp.float32),
        pltpu.VMEM((_CH,), jnp.float32),
        pltpu.VMEM((_CH,), jnp.float32),
        pltpu.SemaphoreType.DMA,
        pltpu.SemaphoreType.DMA,
        pltpu.SemaphoreType.DMA,
    ],
)(_dog_body)


def kernel(images, etf):
    b, ch, y, x = images.shape
    img2 = images.reshape(b, y * x)
    etf3 = etf.reshape(b, 2, y * x)
    out = _dog_call(img2, etf3)
    return out.reshape(b, ch, y, x)


# speedup vs baseline: 1.0068x; 1.0068x over previous
import functools
import math

import jax
import jax.numpy as jnp
from jax import lax
from jax.experimental import pallas as pl
from jax.experimental.pallas import tpu as pltpu
from jax.experimental.pallas import tpu_sc as plsc

_SIGMA_C = 1.0
_RHO = 0.99
_SIGMA_S = _SIGMA_C * 1.6
_MAX_T = math.ceil(_SIGMA_S * 3)


def _gauss(x, sigma):
    return math.exp(-x * x / (2.0 * sigma * sigma)) / (math.sqrt(2.0 * math.pi) * sigma)


_W = {t: _gauss(t, _SIGMA_C) - _RHO * _gauss(t, _SIGMA_S)
      for t in range(-_MAX_T, _MAX_T + 1)}
_TOTAL_W = sum(_W.values())
_TAPS = [t for t in range(-_MAX_T, _MAX_T + 1) if t != 0]
_NT = len(_TAPS)

_B, _Y, _X = 16, 512, 512
_N = _Y * _X
_NC, _NS, _L = 2, 16, 16
_PX = _N // _NS
_CH = 2048
_NV = _CH // _L
_GS = 128
_NG = _CH // _GS
_MAGIC = 2.0 ** 23
_UNROLL = 4


def _dog_body(img_hbm, etf_hbm, out_hbm,
              plane_sh, ety_v, etx_v, iyf_v, ixf_v,
              idx_a, idx_b, val_a, val_b, val_c, val_0, acc_v,
              sem_a, sem_b, sem_0):
    c = lax.axis_index("c")
    s = lax.axis_index("s")
    nb = _B // _NC
    idx_bufs, val_bufs, sems = [idx_a, idx_b], [val_a, val_b, val_c], [sem_a, sem_b]

    def batch_body(k, carry):
        b = c * nb + k
        pltpu.sync_copy(img_hbm.at[b, pl.ds(s * _PX, _PX)],
                        plane_sh.at[pl.ds(s * _PX, _PX)])
        plsc.subcore_barrier()

        def chunk_body(ch, carry2):
            base = s * _PX + ch * _CH
            pltpu.sync_copy(etf_hbm.at[b, 1, pl.ds(base, _CH)], ety_v)
            pltpu.sync_copy(etf_hbm.at[b, 0, pl.ds(base, _CH)], etx_v)

            @plsc.parallel_loop(0, _NV, 1, unroll=_UNROLL)
            def coord_body(v):
                sl = pl.ds(v * _L, _L)
                p = base + v * _L + lax.iota(jnp.int32, _L)
                iyf_v[sl] = jnp.right_shift(p, 9).astype(jnp.float32)
                ixf_v[sl] = jnp.bitwise_and(p, _X - 1).astype(jnp.float32)

            def tap_idx(v, t, idx_ref):
                sl = pl.ds(v * _L, _L)
                tf = jnp.float32(t)
                mg = jnp.float32(_MAGIC)
                py = iyf_v[sl] - ety_v[sl] * tf
                px = ixf_v[sl] + etx_v[sl] * tf
                py = jnp.minimum(jnp.maximum(py, jnp.float32(0.0)),
                                 jnp.float32(_Y - 1))
                px = jnp.minimum(jnp.maximum(px, jnp.float32(0.0)),
                                 jnp.float32(_X - 1))
                pyr = (py + mg) - mg
                pxr = (px + mg) - mg
                flat = pyr * jnp.float32(_X) + pxr
                idx_ref[sl] = flat.astype(jnp.int32)

            def compute_idx(t, idx_ref):
                @plsc.parallel_loop(0, _NV, 1, unroll=_UNROLL)
                def _(v):
                    tap_idx(v, t, idx_ref)

            def fire(idx_ref, val_ref, sem):
                return [pltpu.async_copy(
                    plane_sh.at[idx_ref.at[pl.ds(j * _GS, _GS)]],
                    val_ref.at[pl.ds(j * _GS, _GS)], sem)
                    for j in range(_NG)]

            cp0 = pltpu.async_copy(plane_sh.at[pl.ds(base, _CH)], val_0, sem_0)

            compute_idx(_TAPS[0], idx_bufs[0])
            inflight = {0: fire(idx_bufs[0], val_bufs[0], sems[0])}
            compute_idx(_TAPS[1], idx_bufs[1])
            inflight[1] = fire(idx_bufs[1], val_bufs[1], sems[1])

            for kk in range(_NT):
                for cp in inflight.pop(kk % 2):
                    cp.wait()
                w = jnp.float32(_W[_TAPS[kk]] / _TOTAL_W)
                val_ref = val_bufs[kk % 3]
                has_next = kk + 2 < _NT
                last = kk == _NT - 1
                if last:
                    cp0.wait()
                w0 = jnp.float32(_W[0] / _TOTAL_W)

                @plsc.parallel_loop(0, _NV, 1, unroll=_UNROLL)
                def merged(v):
                    sl = pl.ds(v * _L, _L)
                    if has_next:
                        tap_idx(v, _TAPS[kk + 2], idx_bufs[kk % 2])
                    upd = val_ref[sl] * w
                    if kk == 0:
                        acc_v[sl] = upd
                    elif last:
                        acc_v[sl] = acc_v[sl] + upd + val_0[sl] * w0
                    else:
                        acc_v[sl] = acc_v[sl] + upd

                if has_next:
                    inflight[kk % 2] = fire(idx_bufs[kk % 2],
                                            val_bufs[(kk + 2) % 3],
                                            sems[kk % 2])

            pltpu.sync_copy(acc_v, out_hbm.at[b, pl.ds(base, _CH)])
            return 0

        lax.fori_loop(0, _PX // _CH, chunk_body, 0)
        plsc.subcore_barrier()
        return 0

    lax.fori_loop(0, nb, batch_body, 0)


_dog_call = functools.partial(
    pl.kernel,
    out_type=jax.ShapeDtypeStruct((_B, _N), jnp.float32),
    mesh=plsc.VectorSubcoreMesh(core_axis_name="c", subcore_axis_name="s"),
    scratch_types=[
        pltpu.VMEM_SHARED((_N,), jnp.float32),
        pltpu.VMEM((_CH,), jnp.float32),
        pltpu.VMEM((_CH,), jnp.float32),
        pltpu.VMEM((_CH,), jnp.float32),
        pltpu.VMEM((_CH,), jnp.float32),
        pltpu.VMEM((_CH,), jnp.int32),
        pltpu.VMEM((_CH,), jnp.int32),
        pltpu.VMEM((_CH,), jnp.float32),
        pltpu.VMEM((_CH,), jnp.float32),
        pltpu.VMEM((_CH,), jnp.float32),
        pltpu.VMEM((_CH,), jnp.float32),
        pltpu.VMEM((_CH,), jnp.float32),
        pltpu.SemaphoreType.DMA,
        pltpu.SemaphoreType.DMA,
        pltpu.SemaphoreType.DMA,
    ],
)(_dog_body)


def kernel(images, etf):
    b, ch, y, x = images.shape
    img2 = images.reshape(b, y * x)
    etf3 = etf.reshape(b, 2, y * x)
    out = _dog_call(img2, etf3)
    return out.reshape(b, ch, y, x)
